# Initial kernel scaffold; baseline (speedup 1.0000x reference)
import jax
import jax.numpy as jnp
from jax.experimental import pallas as pl


def kernel(hidden_states, positions, W, b):
    raise NotImplementedError("write your pallas kernel here")


# speedup vs baseline: 2.3096x; 2.3096x over previous
import jax
import jax.numpy as jnp
from jax.experimental import pallas as pl

_B, _N, _D = 64, 32, 128
_G = 8
_GG = _G * _G
_NH = 4.0
_CELL = _NH / _G
_KB = 4
_M = _KB * _N
_NBLK = _B // _KB


def _body(h_ref, pxr_ref, pxc_ref, pyr_ref, pyc_ref, w_ref, b_ref, out_ref):
    h = h_ref[...]
    relx = pxr_ref[...] - pxc_ref[...]
    rely = pyr_ref[...] - pyc_ref[...]
    gx = jnp.clip(((relx + _NH / 2.0) / _CELL).astype(jnp.int32), 0, _G - 1)
    gy = jnp.clip(((rely + _NH / 2.0) / _CELL).astype(jnp.int32), 0, _G - 1)
    cell = gx * _G + gy
    ii = jax.lax.broadcasted_iota(jnp.int32, (_M, _M), 0)
    jj = jax.lax.broadcasted_iota(jnp.int32, (_M, _M), 1)
    valid = (ii // _N == jj // _N) & (ii != jj)
    cell = jnp.where(valid, cell, -1)
    hw = jnp.dot(h, w_ref[...], preferred_element_type=jnp.float32)
    acc = jnp.zeros((_M, _D), jnp.float32)
    for g in range(_GG):
        s = (cell == g).astype(jnp.float32)
        acc = acc + jnp.dot(s, hw[:, g * _D:(g + 1) * _D],
                            preferred_element_type=jnp.float32)
    out_ref[...] = acc + b_ref[...]


def kernel(hidden_states, positions, W, b):
    h2 = hidden_states.reshape(_B * _N, _D)
    px = positions[..., 0].reshape(-1)
    py = positions[..., 1].reshape(-1)
    pxr, pxc = px.reshape(1, -1), px.reshape(-1, 1)
    pyr, pyc = py.reshape(1, -1), py.reshape(-1, 1)
    wr = W.reshape(_D, _GG, _D).transpose(2, 1, 0).reshape(_D, _GG * _D)
    b2 = b.reshape(1, _D)
    out = pl.pallas_call(
        _body,
        grid=(_NBLK,),
        in_specs=[
            pl.BlockSpec((_M, _D), lambda i: (i, 0)),
            pl.BlockSpec((1, _M), lambda i: (0, i)),
            pl.BlockSpec((_M, 1), lambda i: (i, 0)),
            pl.BlockSpec((1, _M), lambda i: (0, i)),
            pl.BlockSpec((_M, 1), lambda i: (i, 0)),
            pl.BlockSpec((_D, _GG * _D), lambda i: (0, 0)),
            pl.BlockSpec((1, _D), lambda i: (0, 0)),
        ],
        out_specs=pl.BlockSpec((_M, _D), lambda i: (i, 0)),
        out_shape=jax.ShapeDtypeStruct((_B * _N, _D), jnp.float32),
    )(h2, pxr, pxc, pyr, pyc, wr, b2)
    return out.reshape(_B, _N, _D)


# speedup vs baseline: 2.6197x; 1.1343x over previous
import jax
import jax.numpy as jnp
from jax.experimental import pallas as pl

_B, _N, _D = 64, 32, 128
_G = 8
_GG = _G * _G
_NH = 4.0
_CELL = _NH / _G
_KB = 4
_M = _KB * _N
_NBLK = _B // _KB


def _body(h_ref, pxr_ref, pxc_ref, pyr_ref, pyc_ref, w_ref, b_ref, out_ref):
    h = h_ref[...]
    relx = pxr_ref[...] - pxc_ref[...]
    rely = pyr_ref[...] - pyc_ref[...]
    gx = jnp.clip(((relx + _NH / 2.0) / _CELL).astype(jnp.int32), 0, _G - 1)
    gy = jnp.clip(((rely + _NH / 2.0) / _CELL).astype(jnp.int32), 0, _G - 1)
    cell = gx * _G + gy
    ii = jax.lax.broadcasted_iota(jnp.int32, (_M, _M), 0)
    jj = jax.lax.broadcasted_iota(jnp.int32, (_M, _M), 1)
    valid = (ii // _N == jj // _N) & (ii != jj)
    cell = jnp.where(valid, cell, -1)
    hw = jnp.dot(h.astype(jnp.bfloat16), w_ref[...],
                 preferred_element_type=jnp.float32)
    acc = jnp.zeros((_M, _D), jnp.float32)
    for g in range(_GG):
        s = (cell == g).astype(jnp.float32)
        acc = acc + jnp.dot(s, hw[:, g * _D:(g + 1) * _D],
                            preferred_element_type=jnp.float32)
    out_ref[...] = acc + b_ref[...]


def kernel(hidden_states, positions, W, b):
    h2 = hidden_states.reshape(_B * _N, _D)
    px = positions[..., 0].reshape(-1)
    py = positions[..., 1].reshape(-1)
    pxr, pxc = px.reshape(1, -1), px.reshape(-1, 1)
    pyr, pyc = py.reshape(1, -1), py.reshape(-1, 1)
    wr = W.reshape(_D, _GG, _D).transpose(2, 1, 0).reshape(_D, _GG * _D)
    wr = wr.astype(jnp.bfloat16)
    b2 = b.reshape(1, _D)
    out = pl.pallas_call(
        _body,
        grid=(_NBLK,),
        in_specs=[
            pl.BlockSpec((_M, _D), lambda i: (i, 0)),
            pl.BlockSpec((1, _M), lambda i: (0, i)),
            pl.BlockSpec((_M, 1), lambda i: (i, 0)),
            pl.BlockSpec((1, _M), lambda i: (0, i)),
            pl.BlockSpec((_M, 1), lambda i: (i, 0)),
            pl.BlockSpec((_D, _GG * _D), lambda i: (0, 0)),
            pl.BlockSpec((1, _D), lambda i: (0, 0)),
        ],
        out_specs=pl.BlockSpec((_M, _D), lambda i: (i, 0)),
        out_shape=jax.ShapeDtypeStruct((_B * _N, _D), jnp.float32),
    )(h2, pxr, pxc, pyr, pyc, wr, b2)
    return out.reshape(_B, _N, _D)
